# baseline (speedup 1.0000x reference)
import functools

import jax
import jax.numpy as jnp
from jax import lax
from jax.experimental import pallas as pl
from jax.experimental.pallas import tpu as pltpu
from jax.experimental.pallas import tpu_sc as plsc

N_U = 10000
N_V = 10000
D_IN = 128
HID = 128
OUT = 64
E_DM = 320000
E_LBL = 50000

DA = 144
NG = DA // 16
CH = 128
SCH = 88
NS = 16
NJ = 228
EP = NJ * NS * SCH
ACC_ROWS = N_U + 16
ZCH = 200
NZ = N_U // ZCH

_mesh = plsc.VectorSubcoreMesh(core_axis_name="c", subcore_axis_name="s")

_DN = lax.GatherDimensionNumbers(
    offset_dims=(), collapsed_slice_dims=(0,), start_index_map=(0,))


def _seg_body(x_aug, zeros_in, src_dm, dst_dm, w_dm, src_mm, dst_mm,
              agg_u, agg_v, acc,
              sidx0, sidx1, didx0, didx1, wch0, wch1,
              rin0, rin1, rout0,
              semi0, semi1, semd0, semd1, semg0, semg1, sems0):
    cid = lax.axis_index("c")
    sid = lax.axis_index("s")
    iota16 = lax.iota(jnp.int32, 16)
    cols = [f * 16 + iota16 for f in range(NG)]
    sidx = [sidx0, sidx1]
    didx = [didx0, didx1]
    wch = [wch0, wch1]
    rin = [rin0, rin1]
    rout = [rout0, rout0]
    semi = [semi0, semi1]
    semd = [semd0, semd1]
    semg = [semg0, semg1]
    sems = [sems0, sems0]

    def zchunk(c, carry):
        pltpu.sync_copy(zeros_in, acc.at[pl.ds(c * ZCH, ZCH)])
        return carry
    lax.fori_loop(0, NZ // NS, lambda j, cr: zchunk(sid + NS * j, cr), 0)

    @pl.when(sid < NZ % NS)
    def _ztail():
        zchunk(sid + NS * (NZ // NS), 0)

    @pl.when(sid == 0)
    def _zdummy():
        pltpu.sync_copy(zeros_in.at[pl.ds(0, 16)], acc.at[pl.ds(N_U, 16)])
    plsc.subcore_barrier()

    def make_pipeline(src_e, dst_e, w_e):
        weighted = w_e is not None

        def idxs_issue(s, k):
            sl = pl.ds((sid + NS * k) * SCH, SCH)
            pltpu.async_copy(src_e.at[sl], sidx[s], semi[s])
            if weighted:
                pltpu.async_copy(w_e.at[sl], wch[s], semi[s])

        def idxs_wait(s):
            pltpu.make_async_copy(src_e.at[pl.ds(0, SCH)], sidx[s],
                                  semi[s]).wait()
            if weighted:
                pltpu.make_async_copy(w_e.at[pl.ds(0, SCH)], wch[s],
                                      semi[s]).wait()

        def didx_issue(s, k):
            sl = pl.ds((sid + NS * k) * SCH, SCH)
            pltpu.async_copy(dst_e.at[sl], didx[s], semd[s])

        def didx_wait(s):
            pltpu.make_async_copy(dst_e.at[pl.ds(0, SCH)], didx[s],
                                  semd[s]).wait()

        def g_issue(s):
            pltpu.async_copy(x_aug.at[sidx[s]], rin[s], semg[s])

        def g_wait(s):
            pltpu.make_async_copy(x_aug.at[sidx[s]], rin[s], semg[s]).wait()

        sc_src = rout if weighted else rin

        def sc_issue(s):
            pltpu.async_copy(sc_src[s], acc.at[didx[s]], sems[s], add=True)

        def sc_wait(s):
            pltpu.make_async_copy(sc_src[s], acc.at[didx[s]], sems[s]).wait()

        def compute(s):
            if not weighted:
                return

            @plsc.parallel_loop(0, SCH, unroll=4)
            def rowloop(e):
                eidx = jnp.full((16,), e, jnp.int32)
                b = plsc.load_gather(wch[s], [eidx])
                for f in range(NG):
                    v = plsc.load_gather(rin[s], [eidx, cols[f]])
                    plsc.store_scatter(rout[s], [eidx, cols[f]], v * b)

        def step(k, s, *, first_own=False, first_other=False,
                 issue_next=True, issue_idx2=True):
            g_wait(s)
            if issue_next:
                idxs_wait(1 - s)
                if not weighted and not first_other:
                    sc_wait(1 - s)
                g_issue(1 - s)
            if weighted and not first_own:
                sc_wait(s)
            didx_issue(s, k)
            compute(s)
            if issue_idx2:
                idxs_issue(s, k + 2)
            didx_wait(s)
            sc_issue(s)

        idxs_issue(0, 0)
        idxs_issue(1, 1)
        idxs_wait(0)
        g_issue(0)
        step(0, 0, first_own=True, first_other=True)
        step(jnp.int32(1), 1)

        def pair(jj, cr):
            k = 2 * jj
            step(k, 0)
            step(k + 1, 1)
            return cr
        lax.fori_loop(1, NJ // 2 - 1, pair, 0)

        step(jnp.int32(NJ - 2), 0, issue_idx2=False)
        step(jnp.int32(NJ - 1), 1, issue_next=False, issue_idx2=False)
        sc_wait(0)
        if not weighted:
            sc_wait(1)

    @pl.when(cid == 0)
    def _dm():
        make_pipeline(src_dm, dst_dm, w_dm)

    @pl.when(cid == 1)
    def _mm():
        make_pipeline(src_mm, dst_mm, None)

    plsc.subcore_barrier()

    def ochunk(c, out_ref, carry):
        sl = pl.ds(c * ZCH, ZCH)
        pltpu.sync_copy(acc.at[sl], out_ref.at[sl])
        return carry

    @pl.when(cid == 0)
    def _out_u():
        lax.fori_loop(0, NZ // NS,
                      lambda j, cr: ochunk(sid + NS * j, agg_u, cr), 0)

        @pl.when(sid < NZ % NS)
        def _otail():
            ochunk(sid + NS * (NZ // NS), agg_u, 0)

    @pl.when(cid == 1)
    def _out_v():
        lax.fori_loop(0, NZ // NS,
                      lambda j, cr: ochunk(sid + NS * j, agg_v, cr), 0)

        @pl.when(sid < NZ % NS)
        def _otail():
            ochunk(sid + NS * (NZ // NS), agg_v, 0)


_seg = functools.partial(
    pl.kernel, _seg_body,
    out_type=[jax.ShapeDtypeStruct((N_U, DA), jnp.float32),
              jax.ShapeDtypeStruct((N_V, DA), jnp.float32)],
    mesh=_mesh,
    scratch_types=[
        pltpu.VMEM_SHARED((ACC_ROWS, DA), jnp.float32),
        pltpu.VMEM((SCH,), jnp.int32), pltpu.VMEM((SCH,), jnp.int32),
        pltpu.VMEM((SCH,), jnp.int32), pltpu.VMEM((SCH,), jnp.int32),
        pltpu.VMEM((SCH,), jnp.float32), pltpu.VMEM((SCH,), jnp.float32),
        pltpu.VMEM((SCH, DA), jnp.float32), pltpu.VMEM((SCH, DA), jnp.float32),
        pltpu.VMEM((SCH, DA), jnp.float32),
        pltpu.SemaphoreType.DMA, pltpu.SemaphoreType.DMA,
        pltpu.SemaphoreType.DMA, pltpu.SemaphoreType.DMA,
        pltpu.SemaphoreType.DMA, pltpu.SemaphoreType.DMA,
        pltpu.SemaphoreType.DMA,
    ],
    compiler_params=pltpu.CompilerParams(use_tc_tiling_on_sc=False,
                                         needs_layout_passes=False),
)()


_ROW_BLK = 1000


def _encoder_body(au_ref, x_d_ref, av_ref, x_m_ref,
                  wu1_ref, wus_ref, wu2_ref, wv1_ref, wvs_ref, wv2_ref,
                  zu_ref, zv_ref):
    deg = jnp.maximum(au_ref[:, D_IN:D_IN + 1], 1e-6)
    a_u = au_ref[:, :D_IN] / deg
    h_u = jnp.maximum(
        jnp.dot(a_u, wu1_ref[:], preferred_element_type=jnp.float32)
        + jnp.dot(x_d_ref[:], wus_ref[:], preferred_element_type=jnp.float32),
        0.0)
    zu_ref[:] = jnp.dot(h_u, wu2_ref[:], preferred_element_type=jnp.float32)
    cnt = jnp.maximum(av_ref[:, D_IN:D_IN + 1], 1.0)
    a_v = av_ref[:, :D_IN] / cnt
    h_v = jnp.maximum(
        jnp.dot(a_v, wv1_ref[:], preferred_element_type=jnp.float32)
        + jnp.dot(x_m_ref[:], wvs_ref[:], preferred_element_type=jnp.float32),
        0.0)
    zv_ref[:] = jnp.dot(h_v, wv2_ref[:], preferred_element_type=jnp.float32)


def _encoders(agg_u, x_d, agg_v, x_m, W_u1, W_u_self, W_u2,
              W_v1, W_v_self, W_v2):
    grid = (N_U // _ROW_BLK,)
    aug_spec = pl.BlockSpec((_ROW_BLK, DA), lambda i: (i, 0))
    row_spec = pl.BlockSpec((_ROW_BLK, D_IN), lambda i: (i, 0))
    full = lambda s: pl.BlockSpec(s, lambda i: (0, 0))
    return pl.pallas_call(
        _encoder_body,
        grid=grid,
        in_specs=[aug_spec, row_spec, aug_spec, row_spec,
                  full((D_IN, HID)), full((D_IN, HID)), full((HID, OUT)),
                  full((D_IN, HID)), full((D_IN, HID)), full((HID, OUT))],
        out_specs=[pl.BlockSpec((_ROW_BLK, OUT), lambda i: (i, 0)),
                   pl.BlockSpec((_ROW_BLK, OUT), lambda i: (i, 0))],
        out_shape=[jax.ShapeDtypeStruct((N_U, OUT), jnp.float32),
                   jax.ShapeDtypeStruct((N_V, OUT), jnp.float32)],
    )(agg_u, x_d, agg_v, x_m, W_u1, W_u_self, W_u2, W_v1, W_v_self, W_v2)


NW = 32
DNJ = 13
E_PAD = DNJ * NW * CH


NDS = 4


def _dec_body(zu, zv, e0, e1, out,
              i00, i01, i02, i03, i10, i11, i12, i13,
              r00, r01, r02, r03, r10, r11, r12, r13,
              res0, res1, res2, res3,
              semi0, semi1, semi2, semi3,
              semg00, semg01, semg02, semg03,
              semg10, semg11, semg12, semg13,
              semo0, semo1, semo2, semo3):
    cid = lax.axis_index("c")
    sid = lax.axis_index("s")
    wid = sid * 2 + cid
    iota16 = lax.iota(jnp.int32, 16)
    i0 = [i00, i01, i02, i03]
    i1 = [i10, i11, i12, i13]
    r0 = [r00, r01, r02, r03]
    r1 = [r10, r11, r12, r13]
    res = [res0, res1, res2, res3]
    semi = [semi0, semi1, semi2, semi3]
    semg0 = [semg00, semg01, semg02, semg03]
    semg1 = [semg10, semg11, semg12, semg13]
    semo = [semo0, semo1, semo2, semo3]

    def idx_issue(s, k):
        sl = pl.ds((wid + NW * k) * CH, CH)
        pltpu.async_copy(e0.at[sl], i0[s], semi[s])
        pltpu.async_copy(e1.at[sl], i1[s], semi[s])

    def idx_wait(s):
        pltpu.make_async_copy(e0.at[pl.ds(0, CH)], i0[s], semi[s]).wait()
        pltpu.make_async_copy(e1.at[pl.ds(0, CH)], i1[s], semi[s]).wait()

    def g_issue(s):
        pltpu.async_copy(zu.at[i0[s]], r0[s], semg0[s])
        pltpu.async_copy(zv.at[i1[s]], r1[s], semg1[s])

    def g_wait(s):
        pltpu.make_async_copy(zu.at[i0[s]], r0[s], semg0[s]).wait()
        pltpu.make_async_copy(zv.at[i1[s]], r1[s], semg1[s]).wait()

    def out_issue(s, k):
        pltpu.async_copy(res[s], out.at[pl.ds((wid + NW * k) * CH, CH)],
                         semo[s])

    def out_wait(s):
        pltpu.make_async_copy(res[s], out.at[pl.ds(0, CH)], semo[s]).wait()

    def compute(s):
        def grp(g, gc):
            row = g * 16 + iota16
            acc = jnp.zeros((16,), jnp.float32)
            for f in range(OUT):
                col = jnp.full((16,), f, jnp.int32)
                v0 = plsc.load_gather(r0[s], [row, col])
                v1 = plsc.load_gather(r1[s], [row, col])
                acc = acc + v0 * v1
            res[s][pl.ds(g * 16, 16)] = acc
            return gc
        lax.fori_loop(0, CH // 16, grp, 0)

    for p in range(3):
        idx_issue(p, p)
    for p in range(3):
        idx_wait(p)
        g_issue(p)
    idx_issue(3, 3)
    for k in range(DNJ):
        s = k % NDS
        g_wait(s)
        if k + 3 < DNJ:
            idx_wait((k + 3) % NDS)
            g_issue((k + 3) % NDS)
        if k + 4 < DNJ:
            idx_issue((k + 4) % NDS, k + 4)
        if k >= NDS:
            out_wait(s)
        compute(s)
        out_issue(s, k)
    for p in range(NDS):
        out_wait(p)


_dec = functools.partial(
    pl.kernel, _dec_body,
    out_type=jax.ShapeDtypeStruct((E_PAD,), jnp.float32),
    mesh=_mesh,
    scratch_types=(
        [pltpu.VMEM((CH,), jnp.int32)] * 8
        + [pltpu.VMEM((CH, OUT), jnp.float32)] * 8
        + [pltpu.VMEM((CH,), jnp.float32)] * 4
        + [pltpu.SemaphoreType.DMA] * 16
    ),
    compiler_params=pltpu.CompilerParams(use_tc_tiling_on_sc=False,
                                         needs_layout_passes=False),
)()


def kernel(x_demand, x_measurement, edge_index_dm, edge_index_mm,
           edge_label_index, edge_weight,
           W_u1, W_u_self, W_u2, W_v1, W_v_self, W_v2):
    x_aug = jnp.concatenate(
        [x_measurement,
         jnp.ones((N_V, 1), jnp.float32),
         jnp.zeros((N_V, DA - D_IN - 1), jnp.float32)], axis=1)
    zeros_in = jnp.zeros((ZCH, DA), jnp.float32)
    npad = EP - E_DM
    src_dm = jnp.concatenate([edge_index_dm[0],
                              jnp.zeros((npad,), jnp.int32)])
    dst_dm = jnp.concatenate([edge_index_dm[1],
                              jnp.full((npad,), N_U, jnp.int32)])
    w_dm = jnp.concatenate([edge_weight, jnp.zeros((npad,), jnp.float32)])
    src_mm = jnp.concatenate([edge_index_mm[0],
                              jnp.zeros((npad,), jnp.int32)])
    dst_mm = jnp.concatenate([edge_index_mm[1],
                              jnp.full((npad,), N_V, jnp.int32)])
    agg_u, agg_v = _seg(x_aug, zeros_in, src_dm, dst_dm, w_dm,
                        src_mm, dst_mm)
    z_u, z_v = _encoders(agg_u, x_demand, agg_v, x_measurement,
                         W_u1, W_u_self, W_u2, W_v1, W_v_self, W_v2)
    pad = jnp.zeros((E_PAD - E_LBL,), jnp.int32)
    e0 = jnp.concatenate([edge_label_index[0], pad])
    e1 = jnp.concatenate([edge_label_index[1], pad])
    dots = _dec(z_u, z_v, e0, e1)
    return dots[:E_LBL]


# speedup vs baseline: 1.1065x; 1.1065x over previous
import functools

import jax
import jax.numpy as jnp
from jax import lax
from jax.experimental import pallas as pl
from jax.experimental.pallas import tpu as pltpu
from jax.experimental.pallas import tpu_sc as plsc

N_U = 10000
N_V = 10000
D_IN = 128
HID = 128
OUT = 64
E_DM = 320000
E_LBL = 50000

DA = 144
NG = DA // 16
CH = 128
SCH = 80
NS = 16
NJ = 250
EP = NJ * NS * SCH
ACC_ROWS = N_U + 16
ZCH = 200
NZ = N_U // ZCH

_mesh = plsc.VectorSubcoreMesh(core_axis_name="c", subcore_axis_name="s")

_DN = lax.GatherDimensionNumbers(
    offset_dims=(), collapsed_slice_dims=(0,), start_index_map=(0,))


def _seg_body(x_aug, zeros_in, src_dm, dst_dm, w_dm, src_mm, dst_mm,
              agg_u, agg_v, acc,
              sidx0, sidx1, didx0, didx1, wch0, wch1,
              rin0, rin1, rout0,
              semi0, semi1, semd0, semd1, semg0, semg1, sems0):
    cid = lax.axis_index("c")
    sid = lax.axis_index("s")
    iota16 = lax.iota(jnp.int32, 16)
    cols = [f * 16 + iota16 for f in range(NG)]
    sidx = [sidx0, sidx1]
    didx = [didx0, didx1]
    wch = [wch0, wch1]
    rin = [rin0, rin1]
    rout = [rout0, rout0]
    semi = [semi0, semi1]
    semd = [semd0, semd1]
    semg = [semg0, semg1]
    sems = [sems0, sems0]

    def zchunk(c, carry):
        pltpu.sync_copy(zeros_in, acc.at[pl.ds(c * ZCH, ZCH)])
        return carry
    lax.fori_loop(0, NZ // NS, lambda j, cr: zchunk(sid + NS * j, cr), 0)

    @pl.when(sid < NZ % NS)
    def _ztail():
        zchunk(sid + NS * (NZ // NS), 0)

    @pl.when(sid == 0)
    def _zdummy():
        pltpu.sync_copy(zeros_in.at[pl.ds(0, 16)], acc.at[pl.ds(N_U, 16)])
    plsc.subcore_barrier()

    def make_pipeline(src_e, dst_e, w_e):
        weighted = w_e is not None

        def idxs_issue(s, k):
            sl = pl.ds((sid + NS * k) * SCH, SCH)
            pltpu.async_copy(src_e.at[sl], sidx[s], semi[s])
            if weighted:
                pltpu.async_copy(w_e.at[sl], wch[s], semi[s])

        def idxs_wait(s):
            pltpu.make_async_copy(src_e.at[pl.ds(0, SCH)], sidx[s],
                                  semi[s]).wait()
            if weighted:
                pltpu.make_async_copy(w_e.at[pl.ds(0, SCH)], wch[s],
                                      semi[s]).wait()

        def didx_issue(s, k):
            sl = pl.ds((sid + NS * k) * SCH, SCH)
            pltpu.async_copy(dst_e.at[sl], didx[s], semd[s])

        def didx_wait(s):
            pltpu.make_async_copy(dst_e.at[pl.ds(0, SCH)], didx[s],
                                  semd[s]).wait()

        def g_issue(s):
            pltpu.async_copy(x_aug.at[sidx[s]], rin[s], semg[s])

        def g_wait(s):
            pltpu.make_async_copy(x_aug.at[sidx[s]], rin[s], semg[s]).wait()

        sc_src = rout if weighted else rin

        def sc_issue(s):
            pltpu.async_copy(sc_src[s], acc.at[didx[s]], sems[s], add=True)

        def sc_wait(s):
            pltpu.make_async_copy(sc_src[s], acc.at[didx[s]], sems[s]).wait()

        def compute(s):
            if not weighted:
                return

            @plsc.parallel_loop(0, SCH, unroll=4)
            def rowloop(e):
                eidx = jnp.full((16,), e, jnp.int32)
                b = plsc.load_gather(wch[s], [eidx])
                for f in range(NG):
                    v = plsc.load_gather(rin[s], [eidx, cols[f]])
                    plsc.store_scatter(rout[s], [eidx, cols[f]], v * b)

        def step(k, s, *, first_own=False, first_other=False,
                 issue_next=True, issue_idx2=True):
            g_wait(s)
            if issue_next:
                idxs_wait(1 - s)
                if not weighted and not first_other:
                    sc_wait(1 - s)
                g_issue(1 - s)
            if weighted and not first_own:
                sc_wait(s)
            didx_issue(s, k)
            compute(s)
            if issue_idx2:
                idxs_issue(s, k + 2)
            didx_wait(s)
            sc_issue(s)

        idxs_issue(0, 0)
        idxs_issue(1, 1)
        idxs_wait(0)
        g_issue(0)
        step(0, 0, first_own=True, first_other=True)
        step(jnp.int32(1), 1)

        def pair(jj, cr):
            k = 2 * jj
            step(k, 0)
            step(k + 1, 1)
            return cr
        lax.fori_loop(1, NJ // 2 - 1, pair, 0)

        step(jnp.int32(NJ - 2), 0, issue_idx2=False)
        step(jnp.int32(NJ - 1), 1, issue_next=False, issue_idx2=False)
        sc_wait(0)
        if not weighted:
            sc_wait(1)

    @pl.when(cid == 0)
    def _dm():
        make_pipeline(src_dm, dst_dm, w_dm)

    @pl.when(cid == 1)
    def _mm():
        make_pipeline(src_mm, dst_mm, None)

    plsc.subcore_barrier()

    def ochunk(c, out_ref, carry):
        sl = pl.ds(c * ZCH, ZCH)
        pltpu.sync_copy(acc.at[sl], out_ref.at[sl])
        return carry

    @pl.when(cid == 0)
    def _out_u():
        lax.fori_loop(0, NZ // NS,
                      lambda j, cr: ochunk(sid + NS * j, agg_u, cr), 0)

        @pl.when(sid < NZ % NS)
        def _otail():
            ochunk(sid + NS * (NZ // NS), agg_u, 0)

    @pl.when(cid == 1)
    def _out_v():
        lax.fori_loop(0, NZ // NS,
                      lambda j, cr: ochunk(sid + NS * j, agg_v, cr), 0)

        @pl.when(sid < NZ % NS)
        def _otail():
            ochunk(sid + NS * (NZ // NS), agg_v, 0)


_seg = functools.partial(
    pl.kernel, _seg_body,
    out_type=[jax.ShapeDtypeStruct((N_U, DA), jnp.float32),
              jax.ShapeDtypeStruct((N_V, DA), jnp.float32)],
    mesh=_mesh,
    scratch_types=[
        pltpu.VMEM_SHARED((ACC_ROWS, DA), jnp.float32),
        pltpu.VMEM((SCH,), jnp.int32), pltpu.VMEM((SCH,), jnp.int32),
        pltpu.VMEM((SCH,), jnp.int32), pltpu.VMEM((SCH,), jnp.int32),
        pltpu.VMEM((SCH,), jnp.float32), pltpu.VMEM((SCH,), jnp.float32),
        pltpu.VMEM((SCH, DA), jnp.float32), pltpu.VMEM((SCH, DA), jnp.float32),
        pltpu.VMEM((SCH, DA), jnp.float32),
        pltpu.SemaphoreType.DMA, pltpu.SemaphoreType.DMA,
        pltpu.SemaphoreType.DMA, pltpu.SemaphoreType.DMA,
        pltpu.SemaphoreType.DMA, pltpu.SemaphoreType.DMA,
        pltpu.SemaphoreType.DMA,
    ],
    compiler_params=pltpu.CompilerParams(use_tc_tiling_on_sc=False,
                                         needs_layout_passes=False),
)()


_ROW_BLK = 1000


def _encoder_body(au_ref, x_d_ref, av_ref, x_m_ref,
                  wu1_ref, wus_ref, wu2_ref, wv1_ref, wvs_ref, wv2_ref,
                  zu_ref, zv_ref):
    deg = jnp.maximum(au_ref[:, D_IN:D_IN + 1], 1e-6)
    a_u = au_ref[:, :D_IN] / deg
    h_u = jnp.maximum(
        jnp.dot(a_u, wu1_ref[:], preferred_element_type=jnp.float32)
        + jnp.dot(x_d_ref[:], wus_ref[:], preferred_element_type=jnp.float32),
        0.0)
    zu_ref[:] = jnp.dot(h_u, wu2_ref[:], preferred_element_type=jnp.float32)
    cnt = jnp.maximum(av_ref[:, D_IN:D_IN + 1], 1.0)
    a_v = av_ref[:, :D_IN] / cnt
    h_v = jnp.maximum(
        jnp.dot(a_v, wv1_ref[:], preferred_element_type=jnp.float32)
        + jnp.dot(x_m_ref[:], wvs_ref[:], preferred_element_type=jnp.float32),
        0.0)
    zv_ref[:] = jnp.dot(h_v, wv2_ref[:], preferred_element_type=jnp.float32)


def _encoders(agg_u, x_d, agg_v, x_m, W_u1, W_u_self, W_u2,
              W_v1, W_v_self, W_v2):
    grid = (N_U // _ROW_BLK,)
    aug_spec = pl.BlockSpec((_ROW_BLK, DA), lambda i: (i, 0))
    row_spec = pl.BlockSpec((_ROW_BLK, D_IN), lambda i: (i, 0))
    full = lambda s: pl.BlockSpec(s, lambda i: (0, 0))
    return pl.pallas_call(
        _encoder_body,
        grid=grid,
        in_specs=[aug_spec, row_spec, aug_spec, row_spec,
                  full((D_IN, HID)), full((D_IN, HID)), full((HID, OUT)),
                  full((D_IN, HID)), full((D_IN, HID)), full((HID, OUT))],
        out_specs=[pl.BlockSpec((_ROW_BLK, OUT), lambda i: (i, 0)),
                   pl.BlockSpec((_ROW_BLK, OUT), lambda i: (i, 0))],
        out_shape=[jax.ShapeDtypeStruct((N_U, OUT), jnp.float32),
                   jax.ShapeDtypeStruct((N_V, OUT), jnp.float32)],
    )(agg_u, x_d, agg_v, x_m, W_u1, W_u_self, W_u2, W_v1, W_v_self, W_v2)


NW = 32
DNJ = 13
E_PAD = DNJ * NW * CH


NDS = 4


def _dec_body(zu, zv, e0, e1, out,
              i00, i01, i02, i03, i10, i11, i12, i13,
              r00, r01, r02, r03, r10, r11, r12, r13,
              res0, res1, res2, res3,
              semi0, semi1, semi2, semi3,
              semg00, semg01, semg02, semg03,
              semg10, semg11, semg12, semg13,
              semo0, semo1, semo2, semo3):
    cid = lax.axis_index("c")
    sid = lax.axis_index("s")
    wid = sid * 2 + cid
    iota16 = lax.iota(jnp.int32, 16)
    i0 = [i00, i01, i02, i03]
    i1 = [i10, i11, i12, i13]
    r0 = [r00, r01, r02, r03]
    r1 = [r10, r11, r12, r13]
    res = [res0, res1, res2, res3]
    semi = [semi0, semi1, semi2, semi3]
    semg0 = [semg00, semg01, semg02, semg03]
    semg1 = [semg10, semg11, semg12, semg13]
    semo = [semo0, semo1, semo2, semo3]

    def idx_issue(s, k):
        sl = pl.ds((wid + NW * k) * CH, CH)
        pltpu.async_copy(e0.at[sl], i0[s], semi[s])
        pltpu.async_copy(e1.at[sl], i1[s], semi[s])

    def idx_wait(s):
        pltpu.make_async_copy(e0.at[pl.ds(0, CH)], i0[s], semi[s]).wait()
        pltpu.make_async_copy(e1.at[pl.ds(0, CH)], i1[s], semi[s]).wait()

    def g_issue(s):
        pltpu.async_copy(zu.at[i0[s]], r0[s], semg0[s])
        pltpu.async_copy(zv.at[i1[s]], r1[s], semg1[s])

    def g_wait(s):
        pltpu.make_async_copy(zu.at[i0[s]], r0[s], semg0[s]).wait()
        pltpu.make_async_copy(zv.at[i1[s]], r1[s], semg1[s]).wait()

    def out_issue(s, k):
        pltpu.async_copy(res[s], out.at[pl.ds((wid + NW * k) * CH, CH)],
                         semo[s])

    def out_wait(s):
        pltpu.make_async_copy(res[s], out.at[pl.ds(0, CH)], semo[s]).wait()

    def compute(s):
        def grp(g, gc):
            row = g * 16 + iota16
            acc = jnp.zeros((16,), jnp.float32)
            for f in range(OUT):
                col = jnp.full((16,), f, jnp.int32)
                v0 = plsc.load_gather(r0[s], [row, col])
                v1 = plsc.load_gather(r1[s], [row, col])
                acc = acc + v0 * v1
            res[s][pl.ds(g * 16, 16)] = acc
            return gc
        lax.fori_loop(0, CH // 16, grp, 0)

    for p in range(3):
        idx_issue(p, p)
    for p in range(3):
        idx_wait(p)
        g_issue(p)
    idx_issue(3, 3)
    for k in range(DNJ):
        s = k % NDS
        g_wait(s)
        if k + 3 < DNJ:
            idx_wait((k + 3) % NDS)
            g_issue((k + 3) % NDS)
        if k + 4 < DNJ:
            idx_issue((k + 4) % NDS, k + 4)
        if k >= NDS:
            out_wait(s)
        compute(s)
        out_issue(s, k)
    for p in range(NDS):
        out_wait(p)


_dec = functools.partial(
    pl.kernel, _dec_body,
    out_type=jax.ShapeDtypeStruct((E_PAD,), jnp.float32),
    mesh=_mesh,
    scratch_types=(
        [pltpu.VMEM((CH,), jnp.int32)] * 8
        + [pltpu.VMEM((CH, OUT), jnp.float32)] * 8
        + [pltpu.VMEM((CH,), jnp.float32)] * 4
        + [pltpu.SemaphoreType.DMA] * 16
    ),
    compiler_params=pltpu.CompilerParams(use_tc_tiling_on_sc=False,
                                         needs_layout_passes=False),
)()


def kernel(x_demand, x_measurement, edge_index_dm, edge_index_mm,
           edge_label_index, edge_weight,
           W_u1, W_u_self, W_u2, W_v1, W_v_self, W_v2):
    x_aug = jnp.concatenate(
        [x_measurement,
         jnp.ones((N_V, 1), jnp.float32),
         jnp.zeros((N_V, DA - D_IN - 1), jnp.float32)], axis=1)
    zeros_in = jnp.zeros((ZCH, DA), jnp.float32)
    npad = EP - E_DM
    src_dm = jnp.concatenate([edge_index_dm[0],
                              jnp.zeros((npad,), jnp.int32)])
    dst_dm = jnp.concatenate([edge_index_dm[1],
                              jnp.full((npad,), N_U, jnp.int32)])
    w_dm = jnp.concatenate([edge_weight, jnp.zeros((npad,), jnp.float32)])
    src_mm = jnp.concatenate([edge_index_mm[0],
                              jnp.zeros((npad,), jnp.int32)])
    dst_mm = jnp.concatenate([edge_index_mm[1],
                              jnp.full((npad,), N_V, jnp.int32)])
    agg_u, agg_v = _seg(x_aug, zeros_in, src_dm, dst_dm, w_dm,
                        src_mm, dst_mm)
    z_u, z_v = _encoders(agg_u, x_demand, agg_v, x_measurement,
                         W_u1, W_u_self, W_u2, W_v1, W_v_self, W_v2)
    pad = jnp.zeros((E_PAD - E_LBL,), jnp.int32)
    e0 = jnp.concatenate([edge_label_index[0], pad])
    e1 = jnp.concatenate([edge_label_index[1], pad])
    dots = _dec(z_u, z_v, e0, e1)
    return dots[:E_LBL]
